# baseline (speedup 1.0000x reference)
import jax
import jax.numpy as jnp
from jax.experimental import pallas as pl
from jax.experimental.pallas import tpu as pltpu

_DIM = 384
_HEADS = 12
_HD = 32
_NW = 7
_TOPK = 4
_RS = 8
_B = 4
_H = 56
_W = 56


def _stage1(x_ref, w_ref, b_ref, qkv_ref, pq_ref, pk_ref):
    xr = x_ref[0].reshape(_RS * _W, _DIM)
    y = jnp.dot(xr, w_ref[...], preferred_element_type=jnp.float32) + b_ref[...]
    qkv_ref[0] = y.reshape(_RS, _W, 3 * _DIM)
    pm = y.reshape(_RS, _NW, _RS, 3 * _DIM).mean(axis=(0, 2))
    pq_ref[0, 0] = pm[:, :_DIM]
    pk_ref[0, 0] = pm[:, _DIM:2 * _DIM]


def _route(pq_ref, pk_ref, idx_ref):
    nr = _NW * _NW
    qm = pq_ref[0].reshape(nr, _DIM)
    km = pk_ref[0].reshape(nr, _DIM)
    a = jax.lax.dot_general(qm, km, (((1,), (1,)), ((), ())),
                            preferred_element_type=jnp.float32)
    iota = jax.lax.broadcasted_iota(jnp.int32, (nr, nr), 1)
    cols = []
    for _ in range(_TOPK):
        m = jnp.max(a, axis=1, keepdims=True)
        sel = jnp.where(a >= m, iota, nr)
        it = jnp.min(sel, axis=1, keepdims=True)
        cols.append(it)
        a = jnp.where(iota == it, -jnp.inf, a)
    idx4 = jnp.concatenate(cols, axis=1)
    idx_ref[0] = jnp.concatenate(
        [idx4, jnp.zeros((nr, 128 - _TOPK), jnp.int32)], axis=1)


def _region_attn(q, K, V):
    q3 = q.reshape(_HEADS, _HD, 64)
    k3_ = K.reshape(_HEADS, _HD, 4 * 64)
    v3_ = V.reshape(_HEADS, _HD, 4 * 64)
    logits = jax.lax.dot_general(
        q3, k3_, (((1,), (1,)), ((0,), (0,))),
        preferred_element_type=jnp.float32)
    mx = jnp.max(logits, axis=-1, keepdims=True)
    e = jnp.exp(logits - mx)
    p = e / jnp.sum(e, axis=-1, keepdims=True)
    o = jax.lax.dot_general(
        v3_, p, (((2,), (2,)), ((0,), (0,))),
        preferred_element_type=jnp.float32)
    return o.reshape(_DIM, 64)


def _attn(idx_ref, q_ref, k0, k1, k2, k3, v0, v1, v2, v3, o_ref):
    scale = _DIM ** -0.5
    q = q_ref[0, 0] * scale
    K = jnp.concatenate([r[0, 0] for r in (k0, k1, k2, k3)],
                        axis=1)
    V = jnp.concatenate([r[0, 0] for r in (v0, v1, v2, v3)], axis=1)
    o_ref[0, 0] = _region_attn(q, K, V)


def _stage4(at_ref, vp_ref, wl_ref, wo_ref, bo_ref, o_ref):
    i = pl.program_id(1)
    acc = at_ref[0]
    vp = vp_ref[0, pl.ds(i * _RS, _RS + 4), :, :]
    for di in range(5):
        for dj in range(5):
            coef = wl_ref[0, di * 5 + dj]
            acc = acc + coef[None, None, :] * vp[di:di + _RS, dj:dj + _W, :]
    y = jnp.dot(acc.reshape(_RS * _W, _DIM), wo_ref[...],
                preferred_element_type=jnp.float32) + bo_ref[...]
    o_ref[0] = y.reshape(_RS, _W, _DIM)


def kernel(x, Wqkv, bqkv, Wlepe, blepe, Wout, bout):
    f32 = jnp.float32
    nr = _NW * _NW
    W2 = Wqkv.reshape(3 * _DIM, _DIM).T
    b2 = bqkv.reshape(1, 3 * _DIM)

    qkv, pq, pk = pl.pallas_call(
        _stage1,
        grid=(_B, _NW),
        in_specs=[
            pl.BlockSpec((1, _RS, _W, _DIM), lambda b, i: (b, i, 0, 0)),
            pl.BlockSpec((_DIM, 3 * _DIM), lambda b, i: (0, 0)),
            pl.BlockSpec((1, 3 * _DIM), lambda b, i: (0, 0)),
        ],
        out_specs=[
            pl.BlockSpec((1, _RS, _W, 3 * _DIM), lambda b, i: (b, i, 0, 0)),
            pl.BlockSpec((1, 1, _NW, _DIM), lambda b, i: (b, i, 0, 0)),
            pl.BlockSpec((1, 1, _NW, _DIM), lambda b, i: (b, i, 0, 0)),
        ],
        out_shape=[
            jax.ShapeDtypeStruct((_B, _H, _W, 3 * _DIM), f32),
            jax.ShapeDtypeStruct((_B, _NW, _NW, _DIM), f32),
            jax.ShapeDtypeStruct((_B, _NW, _NW, _DIM), f32),
        ],
    )(x.transpose(0, 2, 3, 1), W2, b2)

    idx_pad = pl.pallas_call(
        _route,
        grid=(_B,),
        in_specs=[
            pl.BlockSpec((1, _NW, _NW, _DIM), lambda b: (b, 0, 0, 0)),
            pl.BlockSpec((1, _NW, _NW, _DIM), lambda b: (b, 0, 0, 0)),
        ],
        out_specs=pl.BlockSpec((1, nr, 128), lambda b: (b, 0, 0)),
        out_shape=jax.ShapeDtypeStruct((_B, nr, 128), jnp.int32),
    )(pq, pk)
    idx = idx_pad[:, :, :_TOPK]

    qkv_cs = qkv.reshape(_B, _NW, _RS, _NW, _RS, 3 * _DIM).transpose(
        0, 1, 3, 5, 2, 4).reshape(_B, nr, 3 * _DIM, 64)

    def q_map(b, r, idx_ref):
        return (b, r, 0, 0)

    def kv_map(t, cblk):
        def m(b, r, idx_ref):
            return (b, idx_ref[b, r, t], cblk, 0)
        return m

    in_specs = [pl.BlockSpec((1, 1, _DIM, 64), q_map)]
    for t in range(_TOPK):
        in_specs.append(pl.BlockSpec((1, 1, _DIM, 64), kv_map(t, 1)))
    for t in range(_TOPK):
        in_specs.append(pl.BlockSpec((1, 1, _DIM, 64), kv_map(t, 2)))

    grid_spec = pltpu.PrefetchScalarGridSpec(
        num_scalar_prefetch=1,
        grid=(_B, nr),
        in_specs=in_specs,
        out_specs=pl.BlockSpec((1, 1, _DIM, 64), q_map),
    )
    attn_cs = pl.pallas_call(
        _attn,
        grid_spec=grid_spec,
        out_shape=jax.ShapeDtypeStruct((_B, nr, _DIM, 64), f32),
    )(idx, *([qkv_cs] * 9))

    attn = attn_cs.reshape(_B, _NW, _NW, _DIM, _RS, _RS).transpose(
        0, 1, 4, 2, 5, 3).reshape(_B, _H, _W, _DIM)

    v_pad = jnp.pad(qkv[:, :, :, 2 * _DIM:], ((0, 0), (2, 2), (2, 2), (0, 0)))
    wl = Wlepe.reshape(1, _DIM, 25).transpose(0, 2, 1)
    Wo = Wout.reshape(_DIM, _DIM).T
    bo = bout.reshape(1, _DIM)

    out = pl.pallas_call(
        _stage4,
        grid=(_B, _NW),
        in_specs=[
            pl.BlockSpec((1, _RS, _W, _DIM), lambda b, i: (b, i, 0, 0)),
            pl.BlockSpec((1, _H + 4, _W + 4, _DIM), lambda b, i: (b, 0, 0, 0)),
            pl.BlockSpec((1, 25, _DIM), lambda b, i: (0, 0, 0)),
            pl.BlockSpec((_DIM, _DIM), lambda b, i: (0, 0)),
            pl.BlockSpec((1, _DIM), lambda b, i: (0, 0)),
        ],
        out_specs=pl.BlockSpec((1, _RS, _W, _DIM), lambda b, i: (b, i, 0, 0)),
        out_shape=jax.ShapeDtypeStruct((_B, _H, _W, _DIM), f32),
    )(attn, v_pad, wl, Wo, bo)
    return out.transpose(0, 3, 1, 2)


# speedup vs baseline: 1.4009x; 1.4009x over previous
import jax
import jax.numpy as jnp
from jax.experimental import pallas as pl
from jax.experimental.pallas import tpu as pltpu

_DIM = 384
_HEADS = 12
_HD = 32
_NW = 7
_TOPK = 4
_RS = 8
_B = 4
_H = 56
_W = 56


def _stage1(x_ref, w_ref, b_ref, qkv_ref, vg_ref, pq_ref, pk_ref):
    xr = x_ref[0].reshape(_RS * _W, _DIM)
    y = jnp.dot(xr, w_ref[...], preferred_element_type=jnp.float32) + b_ref[...]
    y4 = y.reshape(_RS, _NW, _RS, 3 * _DIM)
    for j in range(_NW):
        yj = y4[:, j].reshape(64, 3 * _DIM)
        qkv_ref[0, j] = yj.T
    vg_ref[0] = y[:, 2 * _DIM:].reshape(_RS, _W, _DIM)
    pm = y4.mean(axis=(0, 2))
    pq_ref[0, 0] = pm[:, :_DIM]
    pk_ref[0, 0] = pm[:, _DIM:2 * _DIM]


def _route(pq_ref, pk_ref, idx_ref):
    nr = _NW * _NW
    qm = pq_ref[0].reshape(nr, _DIM)
    km = pk_ref[0].reshape(nr, _DIM)
    a = jax.lax.dot_general(qm, km, (((1,), (1,)), ((), ())),
                            preferred_element_type=jnp.float32)
    iota = jax.lax.broadcasted_iota(jnp.int32, (nr, nr), 1)
    cols = []
    for _ in range(_TOPK):
        m = jnp.max(a, axis=1, keepdims=True)
        sel = jnp.where(a >= m, iota, nr)
        it = jnp.min(sel, axis=1, keepdims=True)
        cols.append(it)
        a = jnp.where(iota == it, -jnp.inf, a)
    idx4 = jnp.concatenate(cols, axis=1)
    idx_ref[0] = jnp.concatenate(
        [idx4, jnp.zeros((nr, 128 - _TOPK), jnp.int32)], axis=1)


def _region_attn(q, K, V):
    q3 = q.reshape(_HEADS, _HD, 64)
    k3_ = K.reshape(_HEADS, _HD, 4 * 64)
    v3_ = V.reshape(_HEADS, _HD, 4 * 64)
    logits = jax.lax.dot_general(
        q3, k3_, (((1,), (1,)), ((0,), (0,))),
        preferred_element_type=jnp.float32)
    mx = jnp.max(logits, axis=-1, keepdims=True)
    e = jnp.exp(logits - mx)
    p = e / jnp.sum(e, axis=-1, keepdims=True)
    o = jax.lax.dot_general(
        v3_, p, (((2,), (2,)), ((0,), (0,))),
        preferred_element_type=jnp.float32)
    return o.reshape(_DIM, 64)


def _attn(idx_ref, q_ref, k0, k1, k2, k3, v0, v1, v2, v3, o_ref):
    scale = _DIM ** -0.5
    q = q_ref[0, 0] * scale
    K = jnp.concatenate([r[0, 0] for r in (k0, k1, k2, k3)],
                        axis=1)
    V = jnp.concatenate([r[0, 0] for r in (v0, v1, v2, v3)], axis=1)
    o_ref[0, 0] = _region_attn(q, K, V)


def _stage4(at_ref, vp_ref, wl_ref, wo_ref, bo_ref, o_ref):
    i = pl.program_id(1)
    ajs = [at_ref[0, j].T.reshape(_RS, _RS, _DIM) for j in range(_NW)]
    acc = jnp.stack(ajs, axis=0).transpose(1, 0, 2, 3).reshape(
        _RS, _W, _DIM)
    vp = vp_ref[0, pl.ds(i * _RS, _RS + 4), :, :]
    for di in range(5):
        for dj in range(5):
            coef = wl_ref[0, di * 5 + dj]
            acc = acc + coef[None, None, :] * vp[di:di + _RS, dj:dj + _W, :]
    y = jnp.dot(acc.reshape(_RS * _W, _DIM), wo_ref[...],
                preferred_element_type=jnp.float32) + bo_ref[...]
    o_ref[0] = y.reshape(_RS, _W, _DIM)


def kernel(x, Wqkv, bqkv, Wlepe, blepe, Wout, bout):
    f32 = jnp.float32
    nr = _NW * _NW
    W2 = Wqkv.reshape(3 * _DIM, _DIM).T
    b2 = bqkv.reshape(1, 3 * _DIM)

    qkv_cs, v_sp, pq, pk = pl.pallas_call(
        _stage1,
        grid=(_B, _NW),
        in_specs=[
            pl.BlockSpec((1, _RS, _W, _DIM), lambda b, i: (b, i, 0, 0)),
            pl.BlockSpec((_DIM, 3 * _DIM), lambda b, i: (0, 0)),
            pl.BlockSpec((1, 3 * _DIM), lambda b, i: (0, 0)),
        ],
        out_specs=[
            pl.BlockSpec((1, _NW, 3 * _DIM, 64), lambda b, i: (b, i, 0, 0)),
            pl.BlockSpec((1, _RS, _W, _DIM), lambda b, i: (b, i, 0, 0)),
            pl.BlockSpec((1, 1, _NW, _DIM), lambda b, i: (b, i, 0, 0)),
            pl.BlockSpec((1, 1, _NW, _DIM), lambda b, i: (b, i, 0, 0)),
        ],
        out_shape=[
            jax.ShapeDtypeStruct((_B, nr, 3 * _DIM, 64), f32),
            jax.ShapeDtypeStruct((_B, _H, _W, _DIM), f32),
            jax.ShapeDtypeStruct((_B, _NW, _NW, _DIM), f32),
            jax.ShapeDtypeStruct((_B, _NW, _NW, _DIM), f32),
        ],
    )(x.transpose(0, 2, 3, 1), W2, b2)

    idx_pad = pl.pallas_call(
        _route,
        grid=(_B,),
        in_specs=[
            pl.BlockSpec((1, _NW, _NW, _DIM), lambda b: (b, 0, 0, 0)),
            pl.BlockSpec((1, _NW, _NW, _DIM), lambda b: (b, 0, 0, 0)),
        ],
        out_specs=pl.BlockSpec((1, nr, 128), lambda b: (b, 0, 0)),
        out_shape=jax.ShapeDtypeStruct((_B, nr, 128), jnp.int32),
    )(pq, pk)
    idx = idx_pad[:, :, :_TOPK]

    def q_map(b, r, idx_ref):
        return (b, r, 0, 0)

    def kv_map(t, cblk):
        def m(b, r, idx_ref):
            return (b, idx_ref[b, r, t], cblk, 0)
        return m

    in_specs = [pl.BlockSpec((1, 1, _DIM, 64), q_map)]
    for t in range(_TOPK):
        in_specs.append(pl.BlockSpec((1, 1, _DIM, 64), kv_map(t, 1)))
    for t in range(_TOPK):
        in_specs.append(pl.BlockSpec((1, 1, _DIM, 64), kv_map(t, 2)))

    grid_spec = pltpu.PrefetchScalarGridSpec(
        num_scalar_prefetch=1,
        grid=(_B, nr),
        in_specs=in_specs,
        out_specs=pl.BlockSpec((1, 1, _DIM, 64), q_map),
    )
    attn_cs = pl.pallas_call(
        _attn,
        grid_spec=grid_spec,
        out_shape=jax.ShapeDtypeStruct((_B, nr, _DIM, 64), f32),
    )(idx, *([qkv_cs] * 9))

    v_pad = jnp.pad(v_sp, ((0, 0), (2, 2), (2, 2), (0, 0)))
    wl = Wlepe.reshape(1, _DIM, 25).transpose(0, 2, 1)
    Wo = Wout.reshape(_DIM, _DIM).T
    bo = bout.reshape(1, _DIM)

    out = pl.pallas_call(
        _stage4,
        grid=(_B, _NW),
        in_specs=[
            pl.BlockSpec((1, _NW, _DIM, 64), lambda b, i: (b, i, 0, 0)),
            pl.BlockSpec((1, _H + 4, _W + 4, _DIM), lambda b, i: (b, 0, 0, 0)),
            pl.BlockSpec((1, 25, _DIM), lambda b, i: (0, 0, 0)),
            pl.BlockSpec((_DIM, _DIM), lambda b, i: (0, 0)),
            pl.BlockSpec((1, _DIM), lambda b, i: (0, 0)),
        ],
        out_specs=pl.BlockSpec((1, _RS, _W, _DIM), lambda b, i: (b, i, 0, 0)),
        out_shape=jax.ShapeDtypeStruct((_B, _H, _W, _DIM), f32),
    )(attn_cs, v_pad, wl, Wo, bo)
    return out.transpose(0, 3, 1, 2)


# speedup vs baseline: 1.4226x; 1.0155x over previous
import jax
import jax.numpy as jnp
from jax.experimental import pallas as pl
from jax.experimental.pallas import tpu as pltpu

_DIM = 384
_HEADS = 12
_HD = 32
_NW = 7
_TOPK = 4
_RS = 8
_B = 4
_H = 56
_W = 56


def _stage1(x_ref, w_ref, b_ref, qkv_ref, vg_ref, pq_ref, pk_ref):
    xr = x_ref[0].reshape(_RS * _W, _DIM)
    y = jnp.dot(xr, w_ref[...], preferred_element_type=jnp.float32) + b_ref[...]
    y4 = y.reshape(_RS, _NW, _RS, 3 * _DIM)
    for j in range(_NW):
        yj = y4[:, j].reshape(64, 3 * _DIM)
        qkv_ref[0, j] = yj.T
    vg_ref[0] = y[:, 2 * _DIM:].reshape(_RS, _W, _DIM)
    pm = y4.mean(axis=(0, 2))
    pq_ref[0, 0] = pm[:, :_DIM]
    pk_ref[0, 0] = pm[:, _DIM:2 * _DIM]


def _route(pq_ref, pk_ref, idx_ref):
    nr = _NW * _NW
    qm = pq_ref[0].reshape(nr, _DIM)
    km = pk_ref[0].reshape(nr, _DIM)
    a = jax.lax.dot_general(qm, km, (((1,), (1,)), ((), ())),
                            preferred_element_type=jnp.float32)
    iota = jax.lax.broadcasted_iota(jnp.int32, (nr, nr), 1)
    cols = []
    for _ in range(_TOPK):
        m = jnp.max(a, axis=1, keepdims=True)
        sel = jnp.where(a >= m, iota, nr)
        it = jnp.min(sel, axis=1, keepdims=True)
        cols.append(it)
        a = jnp.where(iota == it, -jnp.inf, a)
    idx4 = jnp.concatenate(cols, axis=1)
    idx_ref[0] = jnp.concatenate(
        [idx4, jnp.zeros((nr, 128 - _TOPK), jnp.int32)], axis=1)


def _region_attn(q, K, V):
    q3 = q.reshape(_HEADS, _HD, 64)
    k3_ = K.reshape(_HEADS, _HD, 4 * 64)
    v3_ = V.reshape(_HEADS, _HD, 4 * 64)
    hh = _HEADS // 2
    outs = []
    for c in range(2):
        sl = slice(c * hh, (c + 1) * hh)
        logits = jax.lax.dot_general(
            q3[sl], k3_[sl], (((1,), (1,)), ((0,), (0,))),
            preferred_element_type=jnp.float32)
        e = jnp.exp(logits)
        p = e / jnp.sum(e, axis=-1, keepdims=True)
        o = jax.lax.dot_general(
            v3_[sl], p, (((2,), (2,)), ((0,), (0,))),
            preferred_element_type=jnp.float32)
        outs.append(o)
    return jnp.concatenate(outs, axis=0).reshape(_DIM, 64)


def _attn(idx_ref, q_ref, k0, k1, k2, k3, v0, v1, v2, v3, o_ref):
    scale = _DIM ** -0.5
    q = q_ref[0, 0] * scale
    K = jnp.concatenate([r[0, 0] for r in (k0, k1, k2, k3)],
                        axis=1)
    V = jnp.concatenate([r[0, 0] for r in (v0, v1, v2, v3)], axis=1)
    o_ref[0, 0] = _region_attn(q, K, V)


def _stage4(at_ref, vp_ref, wl_ref, wo_ref, bo_ref, o_ref):
    i = pl.program_id(1)
    ajs = [at_ref[0, j].T.reshape(_RS, _RS, _DIM) for j in range(_NW)]
    acc = jnp.stack(ajs, axis=0).transpose(1, 0, 2, 3).reshape(
        _RS, _W, _DIM)
    vp = vp_ref[0, pl.ds(i * _RS, _RS + 4), :, :]
    for di in range(5):
        for dj in range(5):
            coef = wl_ref[0, di * 5 + dj]
            acc = acc + coef[None, None, :] * vp[di:di + _RS, dj:dj + _W, :]
    y = jnp.dot(acc.reshape(_RS * _W, _DIM), wo_ref[...],
                preferred_element_type=jnp.float32) + bo_ref[...]
    o_ref[0] = y.reshape(_RS, _W, _DIM)


def kernel(x, Wqkv, bqkv, Wlepe, blepe, Wout, bout):
    f32 = jnp.float32
    nr = _NW * _NW
    W2 = Wqkv.reshape(3 * _DIM, _DIM).T
    b2 = bqkv.reshape(1, 3 * _DIM)

    qkv_cs, v_sp, pq, pk = pl.pallas_call(
        _stage1,
        grid=(_B, _NW),
        in_specs=[
            pl.BlockSpec((1, _RS, _W, _DIM), lambda b, i: (b, i, 0, 0)),
            pl.BlockSpec((_DIM, 3 * _DIM), lambda b, i: (0, 0)),
            pl.BlockSpec((1, 3 * _DIM), lambda b, i: (0, 0)),
        ],
        out_specs=[
            pl.BlockSpec((1, _NW, 3 * _DIM, 64), lambda b, i: (b, i, 0, 0)),
            pl.BlockSpec((1, _RS, _W, _DIM), lambda b, i: (b, i, 0, 0)),
            pl.BlockSpec((1, 1, _NW, _DIM), lambda b, i: (b, i, 0, 0)),
            pl.BlockSpec((1, 1, _NW, _DIM), lambda b, i: (b, i, 0, 0)),
        ],
        out_shape=[
            jax.ShapeDtypeStruct((_B, nr, 3 * _DIM, 64), f32),
            jax.ShapeDtypeStruct((_B, _H, _W, _DIM), f32),
            jax.ShapeDtypeStruct((_B, _NW, _NW, _DIM), f32),
            jax.ShapeDtypeStruct((_B, _NW, _NW, _DIM), f32),
        ],
    )(x.transpose(0, 2, 3, 1), W2, b2)

    idx_pad = pl.pallas_call(
        _route,
        grid=(_B,),
        in_specs=[
            pl.BlockSpec((1, _NW, _NW, _DIM), lambda b: (b, 0, 0, 0)),
            pl.BlockSpec((1, _NW, _NW, _DIM), lambda b: (b, 0, 0, 0)),
        ],
        out_specs=pl.BlockSpec((1, nr, 128), lambda b: (b, 0, 0)),
        out_shape=jax.ShapeDtypeStruct((_B, nr, 128), jnp.int32),
    )(pq, pk)
    idx = idx_pad[:, :, :_TOPK]

    def q_map(b, r, idx_ref):
        return (b, r, 0, 0)

    def kv_map(t, cblk):
        def m(b, r, idx_ref):
            return (b, idx_ref[b, r, t], cblk, 0)
        return m

    in_specs = [pl.BlockSpec((1, 1, _DIM, 64), q_map)]
    for t in range(_TOPK):
        in_specs.append(pl.BlockSpec((1, 1, _DIM, 64), kv_map(t, 1)))
    for t in range(_TOPK):
        in_specs.append(pl.BlockSpec((1, 1, _DIM, 64), kv_map(t, 2)))

    grid_spec = pltpu.PrefetchScalarGridSpec(
        num_scalar_prefetch=1,
        grid=(_B, nr),
        in_specs=in_specs,
        out_specs=pl.BlockSpec((1, 1, _DIM, 64), q_map),
    )
    attn_cs = pl.pallas_call(
        _attn,
        grid_spec=grid_spec,
        out_shape=jax.ShapeDtypeStruct((_B, nr, _DIM, 64), f32),
    )(idx, *([qkv_cs] * 9))

    v_pad = jnp.pad(v_sp, ((0, 0), (2, 2), (2, 2), (0, 0)))
    wl = Wlepe.reshape(1, _DIM, 25).transpose(0, 2, 1)
    Wo = Wout.reshape(_DIM, _DIM).T
    bo = bout.reshape(1, _DIM)

    out = pl.pallas_call(
        _stage4,
        grid=(_B, _NW),
        in_specs=[
            pl.BlockSpec((1, _NW, _DIM, 64), lambda b, i: (b, i, 0, 0)),
            pl.BlockSpec((1, _H + 4, _W + 4, _DIM), lambda b, i: (b, 0, 0, 0)),
            pl.BlockSpec((1, 25, _DIM), lambda b, i: (0, 0, 0)),
            pl.BlockSpec((_DIM, _DIM), lambda b, i: (0, 0)),
            pl.BlockSpec((1, _DIM), lambda b, i: (0, 0)),
        ],
        out_specs=pl.BlockSpec((1, _RS, _W, _DIM), lambda b, i: (b, i, 0, 0)),
        out_shape=jax.ShapeDtypeStruct((_B, _H, _W, _DIM), f32),
    )(attn_cs, v_pad, wl, Wo, bo)
    return out.transpose(0, 3, 1, 2)


# speedup vs baseline: 1.5029x; 1.0564x over previous
import jax
import jax.numpy as jnp
from jax.experimental import pallas as pl
from jax.experimental.pallas import tpu as pltpu

_DIM = 384
_HEADS = 12
_HD = 32
_NW = 7
_TOPK = 4
_RS = 8
_B = 4
_H = 56
_W = 56


def _stage1(x_ref, w_ref, b_ref, qkv_ref, vg_ref, pq_ref, pk_ref):
    xr = x_ref[0].reshape(_RS * _W, _DIM)
    y = jnp.dot(xr, w_ref[...], preferred_element_type=jnp.float32) + b_ref[...]
    y4 = y.reshape(_RS, _NW, _RS, 3 * _DIM)
    for j in range(_NW):
        yj = y4[:, j].reshape(64, 3 * _DIM)
        qkv_ref[0, j] = yj.T
    vg_ref[0] = y[:, 2 * _DIM:].reshape(_RS, _W, _DIM)
    pm = y4.mean(axis=(0, 2))
    pq_ref[0, 0] = pm[:, :_DIM]
    pk_ref[0, 0] = pm[:, _DIM:2 * _DIM]


def _route(pq_ref, pk_ref, idx_ref):
    nr = _NW * _NW
    qm = pq_ref[0].reshape(nr, _DIM)
    km = pk_ref[0].reshape(nr, _DIM)
    a = jax.lax.dot_general(qm, km, (((1,), (1,)), ((), ())),
                            preferred_element_type=jnp.float32)
    iota = jax.lax.broadcasted_iota(jnp.int32, (nr, nr), 1)
    cols = []
    for _ in range(_TOPK):
        m = jnp.max(a, axis=1, keepdims=True)
        sel = jnp.where(a >= m, iota, nr)
        it = jnp.min(sel, axis=1, keepdims=True)
        cols.append(it)
        a = jnp.where(iota == it, -jnp.inf, a)
    idx4 = jnp.concatenate(cols, axis=1)
    idx_ref[0] = jnp.concatenate(
        [idx4, jnp.zeros((nr, 128 - _TOPK), jnp.int32)], axis=1)


def _region_attn(q, K, V):
    q3 = q.reshape(_HEADS, _HD, 64)
    k3_ = K.reshape(_HEADS, _HD, 4 * 64)
    v3_ = V.reshape(_HEADS, _HD, 4 * 64)
    hh = _HEADS // 2
    outs = []
    for c in range(2):
        sl = slice(c * hh, (c + 1) * hh)
        logits = jax.lax.dot_general(
            q3[sl], k3_[sl], (((1,), (1,)), ((0,), (0,))),
            preferred_element_type=jnp.float32)
        e = jnp.exp(logits)
        s = jnp.sum(e, axis=-1, keepdims=True)
        o = jax.lax.dot_general(
            v3_[sl], e, (((2,), (2,)), ((0,), (0,))),
            preferred_element_type=jnp.float32)
        outs.append(o / jnp.swapaxes(s, 1, 2))
    return jnp.concatenate(outs, axis=0).reshape(_DIM, 64)


def _attn(idx_ref, q_ref, k0, k1, k2, k3, v0, v1, v2, v3, o_ref):
    scale = _DIM ** -0.5
    q = q_ref[0, 0] * scale
    K = jnp.concatenate([r[0, 0] for r in (k0, k1, k2, k3)],
                        axis=1)
    V = jnp.concatenate([r[0, 0] for r in (v0, v1, v2, v3)], axis=1)
    o_ref[0, 0] = _region_attn(q, K, V)


def _stage4(at_ref, vp_ref, wl_ref, wo_ref, bo_ref, o_ref):
    i = pl.program_id(1)
    ajs = [at_ref[0, j].T.reshape(_RS, _RS, _DIM) for j in range(_NW)]
    acc = jnp.stack(ajs, axis=0).transpose(1, 0, 2, 3).reshape(
        _RS, _W, _DIM)
    vp = vp_ref[0, pl.ds(i * _RS, _RS + 4), :, :]
    for di in range(5):
        for dj in range(5):
            coef = wl_ref[0, di * 5 + dj]
            acc = acc + coef[None, None, :] * vp[di:di + _RS, dj:dj + _W, :]
    y = jnp.dot(acc.reshape(_RS * _W, _DIM), wo_ref[...],
                preferred_element_type=jnp.float32) + bo_ref[...]
    o_ref[0] = y.reshape(_RS, _W, _DIM)


def kernel(x, Wqkv, bqkv, Wlepe, blepe, Wout, bout):
    f32 = jnp.float32
    nr = _NW * _NW
    W2 = Wqkv.reshape(3 * _DIM, _DIM).T
    b2 = bqkv.reshape(1, 3 * _DIM)

    qkv_cs, v_sp, pq, pk = pl.pallas_call(
        _stage1,
        grid=(_B, _NW),
        in_specs=[
            pl.BlockSpec((1, _RS, _W, _DIM), lambda b, i: (b, i, 0, 0)),
            pl.BlockSpec((_DIM, 3 * _DIM), lambda b, i: (0, 0)),
            pl.BlockSpec((1, 3 * _DIM), lambda b, i: (0, 0)),
        ],
        out_specs=[
            pl.BlockSpec((1, _NW, 3 * _DIM, 64), lambda b, i: (b, i, 0, 0)),
            pl.BlockSpec((1, _RS, _W, _DIM), lambda b, i: (b, i, 0, 0)),
            pl.BlockSpec((1, 1, _NW, _DIM), lambda b, i: (b, i, 0, 0)),
            pl.BlockSpec((1, 1, _NW, _DIM), lambda b, i: (b, i, 0, 0)),
        ],
        out_shape=[
            jax.ShapeDtypeStruct((_B, nr, 3 * _DIM, 64), f32),
            jax.ShapeDtypeStruct((_B, _H, _W, _DIM), f32),
            jax.ShapeDtypeStruct((_B, _NW, _NW, _DIM), f32),
            jax.ShapeDtypeStruct((_B, _NW, _NW, _DIM), f32),
        ],
    )(x.transpose(0, 2, 3, 1), W2, b2)

    idx_pad = pl.pallas_call(
        _route,
        grid=(_B,),
        in_specs=[
            pl.BlockSpec((1, _NW, _NW, _DIM), lambda b: (b, 0, 0, 0)),
            pl.BlockSpec((1, _NW, _NW, _DIM), lambda b: (b, 0, 0, 0)),
        ],
        out_specs=pl.BlockSpec((1, nr, 128), lambda b: (b, 0, 0)),
        out_shape=jax.ShapeDtypeStruct((_B, nr, 128), jnp.int32),
    )(pq, pk)

    def q_map(b, r, idx_ref):
        return (b, r, 0, 0)

    def kv_map(t, cblk):
        def m(b, r, idx_ref):
            return (b, idx_ref[b, r, t], cblk, 0)
        return m

    in_specs = [pl.BlockSpec((1, 1, _DIM, 64), q_map)]
    for t in range(_TOPK):
        in_specs.append(pl.BlockSpec((1, 1, _DIM, 64), kv_map(t, 1)))
    for t in range(_TOPK):
        in_specs.append(pl.BlockSpec((1, 1, _DIM, 64), kv_map(t, 2)))

    grid_spec = pltpu.PrefetchScalarGridSpec(
        num_scalar_prefetch=1,
        grid=(_B, nr),
        in_specs=in_specs,
        out_specs=pl.BlockSpec((1, 1, _DIM, 64), q_map),
    )
    attn_cs = pl.pallas_call(
        _attn,
        grid_spec=grid_spec,
        out_shape=jax.ShapeDtypeStruct((_B, nr, _DIM, 64), f32),
    )(idx_pad, *([qkv_cs] * 9))

    v_pad = jnp.pad(v_sp, ((0, 0), (2, 2), (2, 2), (0, 0)))
    wl = Wlepe.reshape(1, _DIM, 25).transpose(0, 2, 1)
    Wo = Wout.reshape(_DIM, _DIM).T
    bo = bout.reshape(1, _DIM)

    out = pl.pallas_call(
        _stage4,
        grid=(_B, _NW),
        in_specs=[
            pl.BlockSpec((1, _NW, _DIM, 64), lambda b, i: (b, i, 0, 0)),
            pl.BlockSpec((1, _H + 4, _W + 4, _DIM), lambda b, i: (b, 0, 0, 0)),
            pl.BlockSpec((1, 25, _DIM), lambda b, i: (0, 0, 0)),
            pl.BlockSpec((_DIM, _DIM), lambda b, i: (0, 0)),
            pl.BlockSpec((1, _DIM), lambda b, i: (0, 0)),
        ],
        out_specs=pl.BlockSpec((1, _RS, _W, _DIM), lambda b, i: (b, i, 0, 0)),
        out_shape=jax.ShapeDtypeStruct((_B, _H, _W, _DIM), f32),
    )(attn_cs, v_pad, wl, Wo, bo)
    return out.transpose(0, 3, 1, 2)


# speedup vs baseline: 1.5111x; 1.0054x over previous
import jax
import jax.numpy as jnp
from jax.experimental import pallas as pl
from jax.experimental.pallas import tpu as pltpu

_DIM = 384
_HEADS = 12
_HD = 32
_NW = 7
_TOPK = 4
_RS = 8
_B = 4
_H = 56
_W = 56


def _stage1(x_ref, w_ref, b_ref, qkv_ref, vg_ref, pq_ref, pk_ref):
    xr = x_ref[0].reshape(_RS * _W, _DIM).astype(jnp.bfloat16)
    y = jnp.dot(xr, w_ref[...].astype(jnp.bfloat16),
                preferred_element_type=jnp.float32) + b_ref[...]
    y4 = y.reshape(_RS, _NW, _RS, 3 * _DIM)
    for j in range(_NW):
        yj = y4[:, j].reshape(64, 3 * _DIM)
        qkv_ref[0, j] = yj.T
    vg_ref[0] = y[:, 2 * _DIM:].reshape(_RS, _W, _DIM)
    pm = y4.mean(axis=(0, 2))
    pq_ref[0, 0] = pm[:, :_DIM]
    pk_ref[0, 0] = pm[:, _DIM:2 * _DIM]


def _route(pq_ref, pk_ref, idx_ref):
    nr = _NW * _NW
    qm = pq_ref[0].reshape(nr, _DIM)
    km = pk_ref[0].reshape(nr, _DIM)
    a = jax.lax.dot_general(qm, km, (((1,), (1,)), ((), ())),
                            preferred_element_type=jnp.float32)
    iota = jax.lax.broadcasted_iota(jnp.int32, (nr, nr), 1)
    cols = []
    for _ in range(_TOPK):
        m = jnp.max(a, axis=1, keepdims=True)
        sel = jnp.where(a >= m, iota, nr)
        it = jnp.min(sel, axis=1, keepdims=True)
        cols.append(it)
        a = jnp.where(iota == it, -jnp.inf, a)
    idx4 = jnp.concatenate(cols, axis=1)
    idx_ref[0] = jnp.concatenate(
        [idx4, jnp.zeros((nr, 128 - _TOPK), jnp.int32)], axis=1)


def _region_attn(q, K, V):
    q3 = q.reshape(_HEADS, _HD, 64)
    k3_ = K.reshape(_HEADS, _HD, 4 * 64)
    v3_ = V.reshape(_HEADS, _HD, 4 * 64)
    hh = _HEADS // 2
    outs = []
    for c in range(2):
        sl = slice(c * hh, (c + 1) * hh)
        logits = jax.lax.dot_general(
            q3[sl], k3_[sl], (((1,), (1,)), ((0,), (0,))),
            preferred_element_type=jnp.float32)
        e = jnp.exp(logits)
        s = jnp.sum(e, axis=-1, keepdims=True)
        o = jax.lax.dot_general(
            v3_[sl], e, (((2,), (2,)), ((0,), (0,))),
            preferred_element_type=jnp.float32)
        outs.append(o / jnp.swapaxes(s, 1, 2))
    return jnp.concatenate(outs, axis=0).reshape(_DIM, 64)


def _attn(idx_ref, q_ref, k0, k1, k2, k3, v0, v1, v2, v3, o_ref):
    scale = _DIM ** -0.5
    q = q_ref[0, 0] * scale
    K = jnp.concatenate([r[0, 0] for r in (k0, k1, k2, k3)],
                        axis=1)
    V = jnp.concatenate([r[0, 0] for r in (v0, v1, v2, v3)], axis=1)
    o_ref[0, 0] = _region_attn(q, K, V)


def _stage4(at_ref, vp_ref, wl_ref, wo_ref, bo_ref, o_ref):
    i = pl.program_id(1)
    ajs = [at_ref[0, j].T.reshape(_RS, _RS, _DIM) for j in range(_NW)]
    acc = jnp.stack(ajs, axis=0).transpose(1, 0, 2, 3).reshape(
        _RS, _W, _DIM)
    vp = vp_ref[0, pl.ds(i * _RS, _RS + 4), :, :]
    for di in range(5):
        for dj in range(5):
            coef = wl_ref[0, di * 5 + dj]
            acc = acc + coef[None, None, :] * vp[di:di + _RS, dj:dj + _W, :]
    y = jnp.dot(acc.reshape(_RS * _W, _DIM).astype(jnp.bfloat16),
                wo_ref[...].astype(jnp.bfloat16),
                preferred_element_type=jnp.float32) + bo_ref[...]
    o_ref[0] = y.reshape(_RS, _W, _DIM)


def kernel(x, Wqkv, bqkv, Wlepe, blepe, Wout, bout):
    f32 = jnp.float32
    nr = _NW * _NW
    W2 = Wqkv.reshape(3 * _DIM, _DIM).T
    b2 = bqkv.reshape(1, 3 * _DIM)

    qkv_cs, v_sp, pq, pk = pl.pallas_call(
        _stage1,
        grid=(_B, _NW),
        in_specs=[
            pl.BlockSpec((1, _RS, _W, _DIM), lambda b, i: (b, i, 0, 0)),
            pl.BlockSpec((_DIM, 3 * _DIM), lambda b, i: (0, 0)),
            pl.BlockSpec((1, 3 * _DIM), lambda b, i: (0, 0)),
        ],
        out_specs=[
            pl.BlockSpec((1, _NW, 3 * _DIM, 64), lambda b, i: (b, i, 0, 0)),
            pl.BlockSpec((1, _RS, _W, _DIM), lambda b, i: (b, i, 0, 0)),
            pl.BlockSpec((1, 1, _NW, _DIM), lambda b, i: (b, i, 0, 0)),
            pl.BlockSpec((1, 1, _NW, _DIM), lambda b, i: (b, i, 0, 0)),
        ],
        out_shape=[
            jax.ShapeDtypeStruct((_B, nr, 3 * _DIM, 64), f32),
            jax.ShapeDtypeStruct((_B, _H, _W, _DIM), f32),
            jax.ShapeDtypeStruct((_B, _NW, _NW, _DIM), f32),
            jax.ShapeDtypeStruct((_B, _NW, _NW, _DIM), f32),
        ],
    )(x.transpose(0, 2, 3, 1), W2, b2)

    idx_pad = pl.pallas_call(
        _route,
        grid=(_B,),
        in_specs=[
            pl.BlockSpec((1, _NW, _NW, _DIM), lambda b: (b, 0, 0, 0)),
            pl.BlockSpec((1, _NW, _NW, _DIM), lambda b: (b, 0, 0, 0)),
        ],
        out_specs=pl.BlockSpec((1, nr, 128), lambda b: (b, 0, 0)),
        out_shape=jax.ShapeDtypeStruct((_B, nr, 128), jnp.int32),
    )(pq, pk)

    def q_map(b, r, idx_ref):
        return (b, r, 0, 0)

    def kv_map(t, cblk):
        def m(b, r, idx_ref):
            return (b, idx_ref[b, r, t], cblk, 0)
        return m

    in_specs = [pl.BlockSpec((1, 1, _DIM, 64), q_map)]
    for t in range(_TOPK):
        in_specs.append(pl.BlockSpec((1, 1, _DIM, 64), kv_map(t, 1)))
    for t in range(_TOPK):
        in_specs.append(pl.BlockSpec((1, 1, _DIM, 64), kv_map(t, 2)))

    grid_spec = pltpu.PrefetchScalarGridSpec(
        num_scalar_prefetch=1,
        grid=(_B, nr),
        in_specs=in_specs,
        out_specs=pl.BlockSpec((1, 1, _DIM, 64), q_map),
    )
    attn_cs = pl.pallas_call(
        _attn,
        grid_spec=grid_spec,
        out_shape=jax.ShapeDtypeStruct((_B, nr, _DIM, 64), f32),
    )(idx_pad, *([qkv_cs] * 9))

    v_pad = jnp.pad(v_sp, ((0, 0), (2, 2), (2, 2), (0, 0)))
    wl = Wlepe.reshape(1, _DIM, 25).transpose(0, 2, 1)
    Wo = Wout.reshape(_DIM, _DIM).T
    bo = bout.reshape(1, _DIM)

    out = pl.pallas_call(
        _stage4,
        grid=(_B, _NW),
        in_specs=[
            pl.BlockSpec((1, _NW, _DIM, 64), lambda b, i: (b, i, 0, 0)),
            pl.BlockSpec((1, _H + 4, _W + 4, _DIM), lambda b, i: (b, 0, 0, 0)),
            pl.BlockSpec((1, 25, _DIM), lambda b, i: (0, 0, 0)),
            pl.BlockSpec((_DIM, _DIM), lambda b, i: (0, 0)),
            pl.BlockSpec((1, _DIM), lambda b, i: (0, 0)),
        ],
        out_specs=pl.BlockSpec((1, _RS, _W, _DIM), lambda b, i: (b, i, 0, 0)),
        out_shape=jax.ShapeDtypeStruct((_B, _H, _W, _DIM), f32),
    )(attn_cs, v_pad, wl, Wo, bo)
    return out.transpose(0, 3, 1, 2)


# speedup vs baseline: 1.6842x; 1.1145x over previous
import jax
import jax.numpy as jnp
from jax.experimental import pallas as pl
from jax.experimental.pallas import tpu as pltpu

_DIM = 384
_HEADS = 12
_HD = 32
_NW = 7
_TOPK = 4
_RS = 8
_B = 4
_H = 56
_W = 56


def _stage1(x_ref, w_ref, b_ref, qkv_ref, vg_ref, pq_ref, pk_ref):
    xr = x_ref[0].reshape(_RS * _W, _DIM).astype(jnp.bfloat16)
    y = jnp.dot(xr, w_ref[...].astype(jnp.bfloat16),
                preferred_element_type=jnp.float32) + b_ref[...]
    y4 = y.reshape(_RS, _NW, _RS, 3 * _DIM)
    for j in range(_NW):
        yj = y4[:, j].reshape(64, 3 * _DIM)
        qkv_ref[0, j] = yj.T.astype(jnp.bfloat16)
    vg_ref[0] = y[:, 2 * _DIM:].reshape(_RS, _W, _DIM)
    pm = y4.mean(axis=(0, 2))
    pq_ref[0, 0] = pm[:, :_DIM]
    pk_ref[0, 0] = pm[:, _DIM:2 * _DIM]


def _route(pq_ref, pk_ref, idx_ref):
    nr = _NW * _NW
    qm = pq_ref[0].reshape(nr, _DIM)
    km = pk_ref[0].reshape(nr, _DIM)
    a = jax.lax.dot_general(qm, km, (((1,), (1,)), ((), ())),
                            preferred_element_type=jnp.float32)
    iota = jax.lax.broadcasted_iota(jnp.int32, (nr, nr), 1)
    cols = []
    for _ in range(_TOPK):
        m = jnp.max(a, axis=1, keepdims=True)
        sel = jnp.where(a >= m, iota, nr)
        it = jnp.min(sel, axis=1, keepdims=True)
        cols.append(it)
        a = jnp.where(iota == it, -jnp.inf, a)
    idx4 = jnp.concatenate(cols, axis=1)
    idx_ref[0] = jnp.concatenate(
        [idx4, jnp.zeros((nr, 128 - _TOPK), jnp.int32)], axis=1)


def _region_attn(q, K, V):
    scale = _DIM ** -0.5
    q3 = q.reshape(_HEADS, _HD, 64)
    k3_ = K.reshape(_HEADS, _HD, 4 * 64)
    v3_ = V.reshape(_HEADS, _HD, 4 * 64)
    hh = _HEADS // 2
    outs = []
    for c in range(2):
        sl = slice(c * hh, (c + 1) * hh)
        logits = jax.lax.dot_general(
            q3[sl], k3_[sl], (((1,), (1,)), ((0,), (0,))),
            preferred_element_type=jnp.float32)
        e = jnp.exp(logits * scale)
        s = jnp.sum(e, axis=-1, keepdims=True)
        o = jax.lax.dot_general(
            v3_[sl], e.astype(jnp.bfloat16), (((2,), (2,)), ((0,), (0,))),
            preferred_element_type=jnp.float32)
        outs.append(o / jnp.swapaxes(s, 1, 2))
    return jnp.concatenate(outs, axis=0).reshape(_DIM, 64)


def _attn(idx_ref, q_ref, k0, k1, k2, k3, v0, v1, v2, v3, o_ref):
    q = q_ref[0, 0]
    K = jnp.concatenate([r[0, 0] for r in (k0, k1, k2, k3)],
                        axis=1)
    V = jnp.concatenate([r[0, 0] for r in (v0, v1, v2, v3)], axis=1)
    o_ref[0, 0] = _region_attn(q, K, V)


def _stage4(at_ref, vp_ref, wl_ref, wo_ref, bo_ref, o_ref):
    i = pl.program_id(1)
    ajs = [at_ref[0, j].T.reshape(_RS, _RS, _DIM) for j in range(_NW)]
    acc = jnp.stack(ajs, axis=0).transpose(1, 0, 2, 3).reshape(
        _RS, _W, _DIM)
    vp = vp_ref[0, pl.ds(i * _RS, _RS + 4), :, :]
    for di in range(5):
        for dj in range(5):
            coef = wl_ref[0, di * 5 + dj]
            acc = acc + coef[None, None, :] * vp[di:di + _RS, dj:dj + _W, :]
    y = jnp.dot(acc.reshape(_RS * _W, _DIM).astype(jnp.bfloat16),
                wo_ref[...].astype(jnp.bfloat16),
                preferred_element_type=jnp.float32) + bo_ref[...]
    o_ref[0] = y.reshape(_RS, _W, _DIM)


def kernel(x, Wqkv, bqkv, Wlepe, blepe, Wout, bout):
    f32 = jnp.float32
    nr = _NW * _NW
    W2 = Wqkv.reshape(3 * _DIM, _DIM).T
    b2 = bqkv.reshape(1, 3 * _DIM)

    qkv_cs, v_sp, pq, pk = pl.pallas_call(
        _stage1,
        grid=(_B, _NW),
        in_specs=[
            pl.BlockSpec((1, _RS, _W, _DIM), lambda b, i: (b, i, 0, 0)),
            pl.BlockSpec((_DIM, 3 * _DIM), lambda b, i: (0, 0)),
            pl.BlockSpec((1, 3 * _DIM), lambda b, i: (0, 0)),
        ],
        out_specs=[
            pl.BlockSpec((1, _NW, 3 * _DIM, 64), lambda b, i: (b, i, 0, 0)),
            pl.BlockSpec((1, _RS, _W, _DIM), lambda b, i: (b, i, 0, 0)),
            pl.BlockSpec((1, 1, _NW, _DIM), lambda b, i: (b, i, 0, 0)),
            pl.BlockSpec((1, 1, _NW, _DIM), lambda b, i: (b, i, 0, 0)),
        ],
        out_shape=[
            jax.ShapeDtypeStruct((_B, nr, 3 * _DIM, 64), jnp.bfloat16),
            jax.ShapeDtypeStruct((_B, _H, _W, _DIM), f32),
            jax.ShapeDtypeStruct((_B, _NW, _NW, _DIM), f32),
            jax.ShapeDtypeStruct((_B, _NW, _NW, _DIM), f32),
        ],
    )(x.transpose(0, 2, 3, 1), W2, b2)

    idx_pad = pl.pallas_call(
        _route,
        grid=(_B,),
        in_specs=[
            pl.BlockSpec((1, _NW, _NW, _DIM), lambda b: (b, 0, 0, 0)),
            pl.BlockSpec((1, _NW, _NW, _DIM), lambda b: (b, 0, 0, 0)),
        ],
        out_specs=pl.BlockSpec((1, nr, 128), lambda b: (b, 0, 0)),
        out_shape=jax.ShapeDtypeStruct((_B, nr, 128), jnp.int32),
    )(pq, pk)

    def q_map(b, r, idx_ref):
        return (b, r, 0, 0)

    def kv_map(t, cblk):
        def m(b, r, idx_ref):
            return (b, idx_ref[b, r, t], cblk, 0)
        return m

    in_specs = [pl.BlockSpec((1, 1, _DIM, 64), q_map)]
    for t in range(_TOPK):
        in_specs.append(pl.BlockSpec((1, 1, _DIM, 64), kv_map(t, 1)))
    for t in range(_TOPK):
        in_specs.append(pl.BlockSpec((1, 1, _DIM, 64), kv_map(t, 2)))

    grid_spec = pltpu.PrefetchScalarGridSpec(
        num_scalar_prefetch=1,
        grid=(_B, nr),
        in_specs=in_specs,
        out_specs=pl.BlockSpec((1, 1, _DIM, 64), q_map),
    )
    attn_cs = pl.pallas_call(
        _attn,
        grid_spec=grid_spec,
        out_shape=jax.ShapeDtypeStruct((_B, nr, _DIM, 64), f32),
    )(idx_pad, *([qkv_cs] * 9))

    v_pad = jnp.pad(v_sp, ((0, 0), (2, 2), (2, 2), (0, 0)))
    wl = Wlepe.reshape(1, _DIM, 25).transpose(0, 2, 1)
    Wo = Wout.reshape(_DIM, _DIM).T
    bo = bout.reshape(1, _DIM)

    out = pl.pallas_call(
        _stage4,
        grid=(_B, _NW),
        in_specs=[
            pl.BlockSpec((1, _NW, _DIM, 64), lambda b, i: (b, i, 0, 0)),
            pl.BlockSpec((1, _H + 4, _W + 4, _DIM), lambda b, i: (b, 0, 0, 0)),
            pl.BlockSpec((1, 25, _DIM), lambda b, i: (0, 0, 0)),
            pl.BlockSpec((_DIM, _DIM), lambda b, i: (0, 0)),
            pl.BlockSpec((1, _DIM), lambda b, i: (0, 0)),
        ],
        out_specs=pl.BlockSpec((1, _RS, _W, _DIM), lambda b, i: (b, i, 0, 0)),
        out_shape=jax.ShapeDtypeStruct((_B, _H, _W, _DIM), f32),
    )(attn_cs, v_pad, wl, Wo, bo)
    return out.transpose(0, 3, 1, 2)


# speedup vs baseline: 1.7354x; 1.0304x over previous
import jax
import jax.numpy as jnp
from jax.experimental import pallas as pl
from jax.experimental.pallas import tpu as pltpu

_DIM = 384
_HEADS = 12
_HD = 32
_NW = 7
_TOPK = 4
_RS = 8
_B = 4
_H = 56
_W = 56


def _stage1(x_ref, w_ref, b_ref, qkv_ref, vg_ref, pq_ref, pk_ref):
    xr = x_ref[0].reshape(_RS * _W, _DIM).astype(jnp.bfloat16)
    y = jnp.dot(xr, w_ref[...].astype(jnp.bfloat16),
                preferred_element_type=jnp.float32) + b_ref[...]
    y4 = y.reshape(_RS, _NW, _RS, 3 * _DIM)
    for j in range(_NW):
        yj = y4[:, j].reshape(64, 3 * _DIM)
        qkv_ref[0, j] = yj.T.astype(jnp.bfloat16)
    vg_ref[0] = y[:, 2 * _DIM:].reshape(_RS, _W, _DIM).astype(jnp.bfloat16)
    pm = y4.mean(axis=(0, 2))
    pq_ref[0, 0] = pm[:, :_DIM]
    pk_ref[0, 0] = pm[:, _DIM:2 * _DIM]


def _route(pq_ref, pk_ref, idx_ref):
    nr = _NW * _NW
    qm = pq_ref[0].reshape(nr, _DIM)
    km = pk_ref[0].reshape(nr, _DIM)
    a = jax.lax.dot_general(qm, km, (((1,), (1,)), ((), ())),
                            preferred_element_type=jnp.float32)
    iota = jax.lax.broadcasted_iota(jnp.int32, (nr, nr), 1)
    cols = []
    for _ in range(_TOPK):
        m = jnp.max(a, axis=1, keepdims=True)
        sel = jnp.where(a >= m, iota, nr)
        it = jnp.min(sel, axis=1, keepdims=True)
        cols.append(it)
        a = jnp.where(iota == it, -jnp.inf, a)
    idx4 = jnp.concatenate(cols, axis=1)
    idx_ref[0] = jnp.concatenate(
        [idx4, jnp.zeros((nr, 128 - _TOPK), jnp.int32)], axis=1)


def _region_attn(q, K, V):
    scale = _DIM ** -0.5
    q3 = q.reshape(_HEADS, _HD, 64)
    k3_ = K.reshape(_HEADS, _HD, 4 * 64)
    v3_ = V.reshape(_HEADS, _HD, 4 * 64)
    hh = _HEADS // 2
    outs = []
    for c in range(2):
        sl = slice(c * hh, (c + 1) * hh)
        logits = jax.lax.dot_general(
            q3[sl], k3_[sl], (((1,), (1,)), ((0,), (0,))),
            preferred_element_type=jnp.float32)
        e = jnp.exp(logits * scale)
        s = jnp.sum(e, axis=-1, keepdims=True)
        o = jax.lax.dot_general(
            v3_[sl], e.astype(jnp.bfloat16), (((2,), (2,)), ((0,), (0,))),
            preferred_element_type=jnp.float32)
        outs.append(o / jnp.swapaxes(s, 1, 2))
    return jnp.concatenate(outs, axis=0).reshape(_DIM, 64).astype(jnp.bfloat16)


def _attn(idx_ref, q_ref, k0, k1, k2, k3, v0, v1, v2, v3, o_ref):
    q = q_ref[0, 0]
    K = jnp.concatenate([r[0, 0] for r in (k0, k1, k2, k3)],
                        axis=1)
    V = jnp.concatenate([r[0, 0] for r in (v0, v1, v2, v3)], axis=1)
    o_ref[0, 0] = _region_attn(q, K, V)


def _stage4(at_ref, vp_ref, wl_ref, wo_ref, bo_ref, o_ref):
    i = pl.program_id(1)
    ajs = [at_ref[0, j].T.reshape(_RS, _RS, _DIM) for j in range(_NW)]
    acc = jnp.stack(ajs, axis=0).transpose(1, 0, 2, 3).reshape(
        _RS, _W, _DIM).astype(jnp.float32)
    vp = vp_ref[0, pl.ds(i * _RS, _RS + 4), :, :]
    for di in range(5):
        for dj in range(5):
            coef = wl_ref[0, di * 5 + dj]
            acc = acc + coef[None, None, :] * vp[di:di + _RS, dj:dj + _W, :]
    y = jnp.dot(acc.reshape(_RS * _W, _DIM).astype(jnp.bfloat16),
                wo_ref[...].astype(jnp.bfloat16),
                preferred_element_type=jnp.float32) + bo_ref[...]
    o_ref[0] = y.reshape(_RS, _W, _DIM)


def kernel(x, Wqkv, bqkv, Wlepe, blepe, Wout, bout):
    f32 = jnp.float32
    nr = _NW * _NW
    W2 = Wqkv.reshape(3 * _DIM, _DIM).T
    b2 = bqkv.reshape(1, 3 * _DIM)

    qkv_cs, v_sp, pq, pk = pl.pallas_call(
        _stage1,
        grid=(_B, _NW),
        in_specs=[
            pl.BlockSpec((1, _RS, _W, _DIM), lambda b, i: (b, i, 0, 0)),
            pl.BlockSpec((_DIM, 3 * _DIM), lambda b, i: (0, 0)),
            pl.BlockSpec((1, 3 * _DIM), lambda b, i: (0, 0)),
        ],
        out_specs=[
            pl.BlockSpec((1, _NW, 3 * _DIM, 64), lambda b, i: (b, i, 0, 0)),
            pl.BlockSpec((1, _RS, _W, _DIM), lambda b, i: (b, i, 0, 0)),
            pl.BlockSpec((1, 1, _NW, _DIM), lambda b, i: (b, i, 0, 0)),
            pl.BlockSpec((1, 1, _NW, _DIM), lambda b, i: (b, i, 0, 0)),
        ],
        out_shape=[
            jax.ShapeDtypeStruct((_B, nr, 3 * _DIM, 64), jnp.bfloat16),
            jax.ShapeDtypeStruct((_B, _H, _W, _DIM), jnp.bfloat16),
            jax.ShapeDtypeStruct((_B, _NW, _NW, _DIM), f32),
            jax.ShapeDtypeStruct((_B, _NW, _NW, _DIM), f32),
        ],
    )(x.transpose(0, 2, 3, 1), W2, b2)

    idx_pad = pl.pallas_call(
        _route,
        grid=(_B,),
        in_specs=[
            pl.BlockSpec((1, _NW, _NW, _DIM), lambda b: (b, 0, 0, 0)),
            pl.BlockSpec((1, _NW, _NW, _DIM), lambda b: (b, 0, 0, 0)),
        ],
        out_specs=pl.BlockSpec((1, nr, 128), lambda b: (b, 0, 0)),
        out_shape=jax.ShapeDtypeStruct((_B, nr, 128), jnp.int32),
    )(pq, pk)

    def q_map(b, r, idx_ref):
        return (b, r, 0, 0)

    def kv_map(t, cblk):
        def m(b, r, idx_ref):
            return (b, idx_ref[b, r, t], cblk, 0)
        return m

    in_specs = [pl.BlockSpec((1, 1, _DIM, 64), q_map)]
    for t in range(_TOPK):
        in_specs.append(pl.BlockSpec((1, 1, _DIM, 64), kv_map(t, 1)))
    for t in range(_TOPK):
        in_specs.append(pl.BlockSpec((1, 1, _DIM, 64), kv_map(t, 2)))

    grid_spec = pltpu.PrefetchScalarGridSpec(
        num_scalar_prefetch=1,
        grid=(_B, nr),
        in_specs=in_specs,
        out_specs=pl.BlockSpec((1, 1, _DIM, 64), q_map),
    )
    attn_cs = pl.pallas_call(
        _attn,
        grid_spec=grid_spec,
        out_shape=jax.ShapeDtypeStruct((_B, nr, _DIM, 64), jnp.bfloat16),
    )(idx_pad, *([qkv_cs] * 9))

    v_pad = jnp.pad(v_sp, ((0, 0), (2, 2), (2, 2), (0, 0)))
    wl = Wlepe.reshape(1, _DIM, 25).transpose(0, 2, 1)
    Wo = Wout.reshape(_DIM, _DIM).T
    bo = bout.reshape(1, _DIM)

    out = pl.pallas_call(
        _stage4,
        grid=(_B, _NW),
        in_specs=[
            pl.BlockSpec((1, _NW, _DIM, 64), lambda b, i: (b, i, 0, 0)),
            pl.BlockSpec((1, _H + 4, _W + 4, _DIM), lambda b, i: (b, 0, 0, 0)),
            pl.BlockSpec((1, 25, _DIM), lambda b, i: (0, 0, 0)),
            pl.BlockSpec((_DIM, _DIM), lambda b, i: (0, 0)),
            pl.BlockSpec((1, _DIM), lambda b, i: (0, 0)),
        ],
        out_specs=pl.BlockSpec((1, _RS, _W, _DIM), lambda b, i: (b, i, 0, 0)),
        out_shape=jax.ShapeDtypeStruct((_B, _H, _W, _DIM), f32),
    )(attn_cs, v_pad, wl, Wo, bo)
    return out.transpose(0, 3, 1, 2)
